# Initial kernel scaffold; baseline (speedup 1.0000x reference)
import jax
import jax.numpy as jnp
from jax.experimental import pallas as pl


def kernel(simcc_x, simcc_y):
    raise NotImplementedError("write your pallas kernel here")


# speedup vs baseline: 1.0728x; 1.0728x over previous
import dataclasses
import functools

import jax
import jax.numpy as jnp
from jax import lax
from jax.experimental import pallas as pl
from jax.experimental.pallas import tpu as pltpu
from jax.experimental.pallas import tpu_sc as plsc

_L = 16
_R = 32
_SPLIT = 2.0

_IBIG = 2**31 - 1


def _row_max_argmax(load_chunk, nchunks, iota):
    m = load_chunk(0)
    idx = iota
    c = iota + _L
    for j in range(1, nchunks):
        v = load_chunk(j)
        gt = v > m
        m = jnp.where(gt, v, m)
        idx = jnp.where(gt, c, idx)
        c = c + _L
    mx = jnp.max(m)
    amax = jnp.min(jnp.where(m == mx, idx, _IBIG))
    return mx, amax


def _block_body(x_vmem, y_vmem, kx_vmem, ky_vmem, sc_vmem):
    wx = x_vmem.shape[1]
    wy = y_vmem.shape[1]
    iota = lax.iota(jnp.int32, _L)
    zf = jnp.zeros((_L,), jnp.float32)
    for g in range(_R // _L):
        def row_body(i, accs, g=g):
            kxa, kya, sca = accs
            r = g * _L + i
            mx, ax = _row_max_argmax(
                lambda j: x_vmem[r, pl.ds(j * _L, _L)], wx // _L, iota)
            my, ay = _row_max_argmax(
                lambda j: y_vmem[r, pl.ds(j * _L, _L)], wy // _L, iota)
            val = jnp.minimum(mx, my)
            neg = val <= jnp.float32(0.0)
            half = jnp.float32(1.0 / _SPLIT)
            kxs = jnp.where(neg, jnp.float32(-1.0),
                            ax.astype(jnp.float32)) * half
            kys = jnp.where(neg, jnp.float32(-1.0),
                            ay.astype(jnp.float32)) * half
            lane = iota == i
            return (jnp.where(lane, kxs, kxa),
                    jnp.where(lane, kys, kya),
                    jnp.where(lane, val, sca))
        kxa, kya, sca = lax.fori_loop(0, _L, row_body, (zf, zf, zf))
        kx_vmem[g, :] = kxa
        ky_vmem[g, :] = kya
        sc_vmem[g, :] = sca


@functools.partial(jax.jit, static_argnums=())
def _sc_decode(sx, sy):
    nk, wx = sx.shape
    wy = sy.shape[1]
    mesh = plsc.VectorSubcoreMesh(core_axis_name="core",
                                  subcore_axis_name="subcore")
    out_t = [jax.ShapeDtypeStruct((nk // _L, _L), jnp.float32)] * 3
    cp = pltpu.CompilerParams()
    if "needs_layout_passes" in pltpu.CompilerParams.__dataclass_fields__:
        cp = dataclasses.replace(cp, needs_layout_passes=False)

    @functools.partial(pl.kernel, out_type=out_t, mesh=mesh,
                       compiler_params=cp)
    def k(x_hbm, y_hbm, kx_hbm, ky_hbm, sc_hbm):
        pltpu.emit_pipeline(
            _block_body,
            grid=(nk // _R,),
            in_specs=[
                pl.BlockSpec((_R, wx), lambda i: (i, 0)),
                pl.BlockSpec((_R, wy), lambda i: (i, 0)),
            ],
            out_specs=[
                pl.BlockSpec((_R // _L, _L), lambda i: (i, 0)),
                pl.BlockSpec((_R // _L, _L), lambda i: (i, 0)),
                pl.BlockSpec((_R // _L, _L), lambda i: (i, 0)),
            ],
            core_axis_name=("core", "subcore"),
            dimension_semantics=(pltpu.PARALLEL,),
        )(x_hbm, y_hbm, kx_hbm, ky_hbm, sc_hbm)

    return k(sx, sy)


def kernel(simcc_x, simcc_y):
    n, k, wx = simcc_x.shape
    wy = simcc_y.shape[2]
    sx = simcc_x.reshape(n * k, wx)
    sy = simcc_y.reshape(n * k, wy)
    kx, ky, sc = _sc_decode(sx, sy)
    keypoints = jnp.stack(
        [kx.reshape(n * k), ky.reshape(n * k)], axis=-1).reshape(n, k, 2)
    scores = sc.reshape(n, k)
    return (keypoints, scores)


# speedup vs baseline: 1.3229x; 1.2332x over previous
import dataclasses
import functools

import jax
import jax.numpy as jnp
from jax import lax
from jax.experimental import pallas as pl
from jax.experimental.pallas import tpu as pltpu
from jax.experimental.pallas import tpu_sc as plsc

_L = 16
_B = 2
_K = 17
_SPLIT = 2.0

_IBIG = 2**31 - 1


def _row_max_argmax(load_chunk, nchunks, iota):
    m = load_chunk(0)
    idx = iota
    c = iota + _L
    for j in range(1, nchunks):
        v = load_chunk(j)
        gt = v > m
        m = jnp.where(gt, v, m)
        idx = jnp.where(gt, c, idx)
        c = c + _L
    mx = jnp.max(m)
    amax = jnp.min(jnp.where(m == mx, idx, _IBIG))
    return mx, amax


def _block_body(x_vmem, y_vmem, kx_vmem, ky_vmem, sc_vmem):
    wx = x_vmem.shape[2]
    wy = y_vmem.shape[2]
    iota = lax.iota(jnp.int32, _L)
    zf = jnp.zeros((_L,), jnp.float32)
    half = jnp.float32(1.0 / _SPLIT)

    for b in range(_B):
        def row_calc(k, b=b):
            mx, ax = _row_max_argmax(
                lambda j: x_vmem[b, k, pl.ds(j * _L, _L)], wx // _L, iota)
            my, ay = _row_max_argmax(
                lambda j: y_vmem[b, k, pl.ds(j * _L, _L)], wy // _L, iota)
            val = jnp.minimum(mx, my)
            neg = val <= jnp.float32(0.0)
            kxs = jnp.where(neg, jnp.float32(-1.0),
                            ax.astype(jnp.float32)) * half
            kys = jnp.where(neg, jnp.float32(-1.0),
                            ay.astype(jnp.float32)) * half
            return kxs, kys, val

        def row_body(k, accs):
            kxa, kya, sca = accs
            kxs, kys, val = row_calc(k)
            lane = iota == k
            return (jnp.where(lane, kxs, kxa),
                    jnp.where(lane, kys, kya),
                    jnp.where(lane, val, sca))

        kxa, kya, sca = lax.fori_loop(0, _L, row_body, (zf, zf, zf))
        kx_vmem[b, pl.ds(0, _L)] = kxa
        ky_vmem[b, pl.ds(0, _L)] = kya
        sc_vmem[b, pl.ds(0, _L)] = sca

        kxs, kys, val = row_calc(_K - 1)
        lane0 = iota == 0
        kx_vmem[b, pl.ds(_L, _L)] = jnp.where(lane0, kxs, zf)
        ky_vmem[b, pl.ds(_L, _L)] = jnp.where(lane0, kys, zf)
        sc_vmem[b, pl.ds(_L, _L)] = jnp.where(lane0, val, zf)


@jax.jit
def _sc_decode(x3, y3):
    n, k, wx = x3.shape
    wy = y3.shape[2]
    mesh = plsc.VectorSubcoreMesh(core_axis_name="core",
                                  subcore_axis_name="subcore")
    out_t = [jax.ShapeDtypeStruct((n, 2 * _L), jnp.float32)] * 3
    cp = pltpu.CompilerParams(use_tc_tiling_on_sc=True)
    if "needs_layout_passes" in pltpu.CompilerParams.__dataclass_fields__:
        cp = dataclasses.replace(cp, needs_layout_passes=False)

    @functools.partial(pl.kernel, out_type=out_t, mesh=mesh,
                       compiler_params=cp)
    def kern(x_hbm, y_hbm, kx_hbm, ky_hbm, sc_hbm):
        pltpu.emit_pipeline(
            _block_body,
            grid=(n // _B,),
            in_specs=[
                pl.BlockSpec((_B, k, wx), lambda i: (i, 0, 0)),
                pl.BlockSpec((_B, k, wy), lambda i: (i, 0, 0)),
            ],
            out_specs=[
                pl.BlockSpec((_B, 2 * _L), lambda i: (i, 0)),
                pl.BlockSpec((_B, 2 * _L), lambda i: (i, 0)),
                pl.BlockSpec((_B, 2 * _L), lambda i: (i, 0)),
            ],
            core_axis_name=("core", "subcore"),
            dimension_semantics=(pltpu.PARALLEL,),
        )(x_hbm, y_hbm, kx_hbm, ky_hbm, sc_hbm)

    return kern(x3, y3)


def kernel(simcc_x, simcc_y):
    n, k, _ = simcc_x.shape
    kx, ky, sc = _sc_decode(simcc_x, simcc_y)
    keypoints = jnp.stack([kx[:, :k], ky[:, :k]], axis=-1)
    scores = sc[:, :k]
    return (keypoints, scores)


# speedup vs baseline: 3.9074x; 2.9537x over previous
import dataclasses
import functools

import jax
import jax.numpy as jnp
from jax import lax
from jax.experimental import pallas as pl
from jax.experimental.pallas import tpu as pltpu
from jax.experimental.pallas import tpu_sc as plsc

_L = 16
_B = 32
_SPLIT = 2.0

_IBIG = 2**31 - 1


def _row_max_argmax(load_chunk, nchunks, iota):
    m = load_chunk(0)
    idx = iota
    c = iota + _L
    for j in range(1, nchunks):
        v = load_chunk(j)
        gt = v > m
        m = jnp.where(gt, v, m)
        idx = jnp.where(gt, c, idx)
        c = c + _L
    mx = jnp.max(m)
    amax = jnp.min(jnp.where(m == mx, idx, _IBIG))
    return mx, amax


def _block_body(x_vmem, y_vmem, kx_vmem, ky_vmem, sc_vmem):
    wx = x_vmem.shape[2]
    wy = y_vmem.shape[2]
    iota = lax.iota(jnp.int32, _L)
    zf = jnp.zeros((_L,), jnp.float32)
    half = jnp.float32(1.0 / _SPLIT)

    for g in range(_B // _L):
        def row_body(i, accs, g=g):
            kxa, kya, sca = accs
            r = g * _L + i
            mx, ax = _row_max_argmax(
                lambda j: x_vmem[0, r, pl.ds(j * _L, _L)], wx // _L, iota)
            my, ay = _row_max_argmax(
                lambda j: y_vmem[0, r, pl.ds(j * _L, _L)], wy // _L, iota)
            val = jnp.minimum(mx, my)
            neg = val <= jnp.float32(0.0)
            kxs = jnp.where(neg, jnp.float32(-1.0),
                            ax.astype(jnp.float32)) * half
            kys = jnp.where(neg, jnp.float32(-1.0),
                            ay.astype(jnp.float32)) * half
            lane = iota == i
            return (jnp.where(lane, kxs, kxa),
                    jnp.where(lane, kys, kya),
                    jnp.where(lane, val, sca))

        kxa, kya, sca = lax.fori_loop(0, _L, row_body, (zf, zf, zf))
        kx_vmem[g, :] = kxa
        ky_vmem[g, :] = kya
        sc_vmem[g, :] = sca


@jax.jit
def _sc_decode(xt, yt):
    k, n, wx = xt.shape
    wy = yt.shape[2]
    mesh = plsc.VectorSubcoreMesh(core_axis_name="core",
                                  subcore_axis_name="subcore")
    out_t = [jax.ShapeDtypeStruct((k * n // _L, _L), jnp.float32)] * 3
    cp = pltpu.CompilerParams(use_tc_tiling_on_sc=True)
    if "needs_layout_passes" in pltpu.CompilerParams.__dataclass_fields__:
        cp = dataclasses.replace(cp, needs_layout_passes=False)

    nblk = n // _B

    @functools.partial(pl.kernel, out_type=out_t, mesh=mesh,
                       compiler_params=cp)
    def kern(x_hbm, y_hbm, kx_hbm, ky_hbm, sc_hbm):
        pltpu.emit_pipeline(
            _block_body,
            grid=(k, nblk),
            in_specs=[
                pl.BlockSpec((1, _B, wx), lambda kk, i: (kk, i, 0)),
                pl.BlockSpec((1, _B, wy), lambda kk, i: (kk, i, 0)),
            ],
            out_specs=[
                pl.BlockSpec((_B // _L, _L),
                             lambda kk, i: (kk * nblk + i, 0)),
                pl.BlockSpec((_B // _L, _L),
                             lambda kk, i: (kk * nblk + i, 0)),
                pl.BlockSpec((_B // _L, _L),
                             lambda kk, i: (kk * nblk + i, 0)),
            ],
            core_axis_name=("core", "subcore"),
            dimension_semantics=(pltpu.PARALLEL, pltpu.PARALLEL),
        )(x_hbm, y_hbm, kx_hbm, ky_hbm, sc_hbm)

    return kern(xt, yt)


def kernel(simcc_x, simcc_y):
    n, k, _ = simcc_x.shape
    xt = jnp.transpose(simcc_x, (1, 0, 2))
    yt = jnp.transpose(simcc_y, (1, 0, 2))
    kx, ky, sc = _sc_decode(xt, yt)
    kxf = kx.reshape(k, n)
    kyf = ky.reshape(k, n)
    scf = sc.reshape(k, n)
    keypoints = jnp.stack([kxf, kyf], axis=-1).transpose(1, 0, 2)
    scores = scf.transpose(1, 0)
    return (keypoints, scores)


# speedup vs baseline: 4.4839x; 1.1475x over previous
import dataclasses
import functools

import jax
import jax.numpy as jnp
from jax import lax
from jax.experimental import pallas as pl
from jax.experimental.pallas import tpu as pltpu
from jax.experimental.pallas import tpu_sc as plsc

_L = 16
_B = 32
_SPLIT = 2.0

_IBIG = 2**31 - 1


def _row_max_argmax(load_chunk, nchunks, iota):
    def scan_range(j0, j1):
        m = load_chunk(j0)
        idx = iota + j0 * _L
        c = iota + (j0 + 1) * _L
        for j in range(j0 + 1, j1):
            v = load_chunk(j)
            gt = v > m
            m = jnp.where(gt, v, m)
            idx = jnp.where(gt, c, idx)
            c = c + _L
        return m, idx

    h = nchunks // 2
    m1, i1 = scan_range(0, h)
    m2, i2 = scan_range(h, nchunks)
    gt = m2 > m1
    m = jnp.where(gt, m2, m1)
    idx = jnp.where(gt, i2, i1)
    mx = jnp.max(m)
    amax = jnp.min(jnp.where(m == mx, idx, _IBIG))
    return mx, amax


def _block_body(x_vmem, y_vmem, kx_vmem, ky_vmem, sc_vmem):
    wx = x_vmem.shape[2]
    wy = y_vmem.shape[2]
    iota = lax.iota(jnp.int32, _L)
    zf = jnp.zeros((_L,), jnp.float32)
    half = jnp.float32(1.0 / _SPLIT)

    for g in range(_B // _L):
        def row_body(i2, accs, g=g):
            kxa, kya, sca = accs
            for u in range(2):
                i = 2 * i2 + u
                r = g * _L + i
                mx, ax = _row_max_argmax(
                    lambda j: x_vmem[0, r, pl.ds(j * _L, _L)],
                    wx // _L, iota)
                my, ay = _row_max_argmax(
                    lambda j: y_vmem[0, r, pl.ds(j * _L, _L)],
                    wy // _L, iota)
                val = jnp.minimum(mx, my)
                neg = val <= jnp.float32(0.0)
                kxs = jnp.where(neg, jnp.float32(-1.0),
                                ax.astype(jnp.float32)) * half
                kys = jnp.where(neg, jnp.float32(-1.0),
                                ay.astype(jnp.float32)) * half
                lane = iota == i
                kxa = jnp.where(lane, kxs, kxa)
                kya = jnp.where(lane, kys, kya)
                sca = jnp.where(lane, val, sca)
            return (kxa, kya, sca)

        kxa, kya, sca = lax.fori_loop(0, _L // 2, row_body, (zf, zf, zf))
        kx_vmem[g, :] = kxa
        ky_vmem[g, :] = kya
        sc_vmem[g, :] = sca


@jax.jit
def _sc_decode(xt, yt):
    k, n, wx = xt.shape
    wy = yt.shape[2]
    mesh = plsc.VectorSubcoreMesh(core_axis_name="core",
                                  subcore_axis_name="subcore")
    out_t = [jax.ShapeDtypeStruct((k * n // _L, _L), jnp.float32)] * 3
    cp = pltpu.CompilerParams(use_tc_tiling_on_sc=True)
    if "needs_layout_passes" in pltpu.CompilerParams.__dataclass_fields__:
        cp = dataclasses.replace(cp, needs_layout_passes=False)

    nblk = n // _B

    @functools.partial(pl.kernel, out_type=out_t, mesh=mesh,
                       compiler_params=cp)
    def kern(x_hbm, y_hbm, kx_hbm, ky_hbm, sc_hbm):
        pltpu.emit_pipeline(
            _block_body,
            grid=(k, nblk),
            in_specs=[
                pl.BlockSpec((1, _B, wx), lambda kk, i: (kk, i, 0)),
                pl.BlockSpec((1, _B, wy), lambda kk, i: (kk, i, 0)),
            ],
            out_specs=[
                pl.BlockSpec((_B // _L, _L),
                             lambda kk, i: (kk * nblk + i, 0)),
                pl.BlockSpec((_B // _L, _L),
                             lambda kk, i: (kk * nblk + i, 0)),
                pl.BlockSpec((_B // _L, _L),
                             lambda kk, i: (kk * nblk + i, 0)),
            ],
            core_axis_name=("core", "subcore"),
            dimension_semantics=(pltpu.PARALLEL, pltpu.PARALLEL),
        )(x_hbm, y_hbm, kx_hbm, ky_hbm, sc_hbm)

    return kern(xt, yt)


def kernel(simcc_x, simcc_y):
    n, k, _ = simcc_x.shape
    xt = jnp.transpose(simcc_x, (1, 0, 2))
    yt = jnp.transpose(simcc_y, (1, 0, 2))
    kx, ky, sc = _sc_decode(xt, yt)
    kxf = kx.reshape(k, n)
    kyf = ky.reshape(k, n)
    scf = sc.reshape(k, n)
    keypoints = jnp.stack([kxf, kyf], axis=-1).transpose(1, 0, 2)
    scores = scf.transpose(1, 0)
    return (keypoints, scores)


# speedup vs baseline: 5.5507x; 1.2379x over previous
import dataclasses
import functools

import jax
import jax.numpy as jnp
from jax import lax
from jax.experimental import pallas as pl
from jax.experimental.pallas import tpu as pltpu
from jax.experimental.pallas import tpu_sc as plsc

_L = 16
_B = 32
_BT = 256
_SPLIT = 2.0

_IBIG = 2**31 - 1


def _row_max_argmax(load_chunk, nchunks, iota):
    def scan_range(j0, j1):
        m = load_chunk(j0)
        idx = iota + j0 * _L
        c = iota + (j0 + 1) * _L
        for j in range(j0 + 1, j1):
            v = load_chunk(j)
            gt = v > m
            m = jnp.where(gt, v, m)
            idx = jnp.where(gt, c, idx)
            c = c + _L
        return m, idx

    h = nchunks // 2
    m1, i1 = scan_range(0, h)
    m2, i2 = scan_range(h, nchunks)
    gt = m2 > m1
    m = jnp.where(gt, m2, m1)
    idx = jnp.where(gt, i2, i1)
    mx = jnp.max(m)
    amax = jnp.min(jnp.where(m == mx, idx, _IBIG))
    return mx, amax


def _sc_block_body(x_vmem, mx_vmem, ax_vmem):
    wx = x_vmem.shape[2]
    iota = lax.iota(jnp.int32, _L)
    zf = jnp.zeros((_L,), jnp.float32)
    zi = jnp.zeros((_L,), jnp.int32)

    for g in range(_B // _L):
        def row_body(i2, accs, g=g):
            mxa, axa = accs
            for u in range(2):
                i = 2 * i2 + u
                r = g * _L + i
                mx, ax = _row_max_argmax(
                    lambda j: x_vmem[0, r, pl.ds(j * _L, _L)],
                    wx // _L, iota)
                lane = iota == i
                mxa = jnp.where(lane, mx, mxa)
                axa = jnp.where(lane, ax, axa)
            return (mxa, axa)

        mxa, axa = lax.fori_loop(0, _L // 2, row_body, (zf, zi))
        mx_vmem[g, :] = mxa
        ax_vmem[g, :] = axa


@jax.jit
def _sc_decode_x(xt):
    k, n, wx = xt.shape
    mesh = plsc.VectorSubcoreMesh(core_axis_name="core",
                                  subcore_axis_name="subcore")
    out_t = [jax.ShapeDtypeStruct((k * n // _L, _L), jnp.float32),
             jax.ShapeDtypeStruct((k * n // _L, _L), jnp.int32)]
    cp = pltpu.CompilerParams(use_tc_tiling_on_sc=True)
    if "needs_layout_passes" in pltpu.CompilerParams.__dataclass_fields__:
        cp = dataclasses.replace(cp, needs_layout_passes=False)

    nblk = n // _B

    @functools.partial(pl.kernel, out_type=out_t, mesh=mesh,
                       compiler_params=cp)
    def kern(x_hbm, mx_hbm, ax_hbm):
        pltpu.emit_pipeline(
            _sc_block_body,
            grid=(k, nblk),
            in_specs=[
                pl.BlockSpec((1, _B, wx), lambda kk, i: (kk, i, 0)),
            ],
            out_specs=[
                pl.BlockSpec((_B // _L, _L),
                             lambda kk, i: (kk * nblk + i, 0)),
                pl.BlockSpec((_B // _L, _L),
                             lambda kk, i: (kk * nblk + i, 0)),
            ],
            core_axis_name=("core", "subcore"),
            dimension_semantics=(pltpu.PARALLEL, pltpu.PARALLEL),
        )(x_hbm, mx_hbm, ax_hbm)

    return kern(xt)


def _tc_block_body(y_ref, my_ref, ay_ref):
    yb = y_ref[...]
    mx = jnp.max(yb, axis=-1)
    eq = yb == mx[:, :, None]
    ii = lax.broadcasted_iota(jnp.int32, yb.shape, 2)
    ay = jnp.min(jnp.where(eq, ii, _IBIG), axis=-1)
    my_ref[...] = mx
    ay_ref[...] = ay


@jax.jit
def _tc_decode_y(yt):
    k, n, wy = yt.shape
    out_t = [jax.ShapeDtypeStruct((k, n), jnp.float32),
             jax.ShapeDtypeStruct((k, n), jnp.int32)]
    grid = (n // _BT,)
    return pl.pallas_call(
        _tc_block_body,
        grid=grid,
        in_specs=[pl.BlockSpec((k, _BT, wy), lambda i: (0, i, 0))],
        out_specs=[pl.BlockSpec((k, _BT), lambda i: (0, i)),
                   pl.BlockSpec((k, _BT), lambda i: (0, i))],
        out_shape=out_t,
        compiler_params=pltpu.CompilerParams(
            dimension_semantics=("arbitrary",)),
    )(yt)


def kernel(simcc_x, simcc_y):
    n, k, _ = simcc_x.shape
    xt = jnp.transpose(simcc_x, (1, 0, 2))
    yt = jnp.transpose(simcc_y, (1, 0, 2))
    mxo, axo = _sc_decode_x(xt)
    my, ay = _tc_decode_y(yt)
    mx = mxo.reshape(k, n)
    ax = axo.reshape(k, n)
    val = jnp.minimum(mx, my)
    neg = val <= 0.0
    half = jnp.float32(1.0 / _SPLIT)
    kx = jnp.where(neg, jnp.float32(-1.0), ax.astype(jnp.float32)) * half
    ky = jnp.where(neg, jnp.float32(-1.0), ay.astype(jnp.float32)) * half
    keypoints = jnp.stack([kx, ky], axis=-1).transpose(1, 0, 2)
    scores = val.transpose(1, 0)
    return (keypoints, scores)
